# Initial kernel scaffold; baseline (speedup 1.0000x reference)
import jax
import jax.numpy as jnp
from jax.experimental import pallas as pl


def kernel(atomic_number, group_mapping, row_mapping):
    raise NotImplementedError("write your pallas kernel here")


# speedup vs baseline: 8.0350x; 8.0350x over previous
import functools

import jax
import jax.numpy as jnp
from jax import lax
from jax.experimental import pallas as pl
from jax.experimental.pallas import tpu as pltpu
from jax.experimental.pallas import tpu_sc as plsc

N = 100000
D = 24
V = 84

_info = plsc.get_sparse_core_info()
_NC, _NS = _info.num_cores, _info.num_subcores
NW = _NC * _NS
BPW = 3200
CH = 128
NCH = BPW // CH
LAST_BASE = N - BPW


def _table_body(gm_ref, rm_ref, tab_ref):
    g = gm_ref[:]
    r = rm_ref[:]
    j = lax.broadcasted_iota(jnp.int32, (V, D), 1)
    hit = ((j < 18) & (j == g)) | ((j >= 18) & ((j - 18) == r))
    tab_ref[:] = jnp.where(hit, jnp.float32(1.0), jnp.float32(0.0))


def _build_table(gm, rm):
    return pl.pallas_call(
        _table_body,
        out_shape=jax.ShapeDtypeStruct((V, D), jnp.float32),
    )(gm[:V].reshape(V, 1), rm[:V].reshape(V, 1))


_mesh = plsc.VectorSubcoreMesh(core_axis_name="c", subcore_axis_name="s")


@functools.partial(
    pl.kernel,
    mesh=_mesh,
    out_type=jax.ShapeDtypeStruct((N, D), jnp.float32),
    scratch_types=[
        pltpu.VMEM((BPW,), jnp.int32),
        pltpu.VMEM((BPW, D), jnp.float32),
        pltpu.SemaphoreType.DMA,
    ],
    compiler_params=pltpu.CompilerParams(use_tc_tiling_on_sc=False),
)
def _gather(an_hbm, table_hbm, out_hbm, idx_v, rows_v, sem):
    wid = lax.axis_index("s") * _NC + lax.axis_index("c")
    base = jnp.minimum(wid * BPW, LAST_BASE)
    base = pl.multiple_of(base, 32)
    pltpu.sync_copy(an_hbm.at[pl.ds(base, BPW)], idx_v)

    def chunk(j, carry):
        off = pl.multiple_of(j * CH, CH)
        pltpu.async_copy(
            table_hbm.at[idx_v.at[pl.ds(off, CH)]],
            rows_v.at[pl.ds(off, CH)],
            sem,
        ).wait()
        return carry

    lax.fori_loop(0, NCH, chunk, 0)
    pltpu.sync_copy(rows_v, out_hbm.at[pl.ds(base, BPW)])


def kernel(atomic_number, group_mapping, row_mapping):
    table = _build_table(group_mapping, row_mapping)
    return _gather(atomic_number, table)


# speedup vs baseline: 8.0619x; 1.0033x over previous
import functools

import jax
import jax.numpy as jnp
from jax import lax
from jax.experimental import pallas as pl
from jax.experimental.pallas import tpu as pltpu
from jax.experimental.pallas import tpu_sc as plsc

N = 100000
D = 24
V = 84

_info = plsc.get_sparse_core_info()
_NC, _NS = _info.num_cores, _info.num_subcores
NW = _NC * _NS
BPW = 3200
CH = 128
NCH = BPW // CH
LAST_BASE = N - BPW


def _table_body(gm_ref, rm_ref, tab_ref):
    g = gm_ref[:]
    r = rm_ref[:]
    j = lax.broadcasted_iota(jnp.int32, (V, D), 1)
    hit = ((j < 18) & (j == g)) | ((j >= 18) & ((j - 18) == r))
    tab_ref[:] = jnp.where(hit, jnp.float32(1.0), jnp.float32(0.0))


def _build_table(gm, rm):
    return pl.pallas_call(
        _table_body,
        out_shape=jax.ShapeDtypeStruct((V, D), jnp.float32),
    )(gm[:V].reshape(V, 1), rm[:V].reshape(V, 1))


_mesh = plsc.VectorSubcoreMesh(core_axis_name="c", subcore_axis_name="s")


@functools.partial(
    pl.kernel,
    mesh=_mesh,
    out_type=jax.ShapeDtypeStruct((N, D), jnp.float32),
    scratch_types=[
        pltpu.VMEM((BPW,), jnp.int32),
        pltpu.VMEM((BPW, D), jnp.float32),
        pltpu.SemaphoreType.DMA,
    ],
    compiler_params=pltpu.CompilerParams(use_tc_tiling_on_sc=False),
)
def _gather(an_hbm, table_hbm, out_hbm, idx_v, rows_v, sem):
    wid = lax.axis_index("s") * _NC + lax.axis_index("c")
    base = jnp.minimum(wid * BPW, LAST_BASE)
    base = pl.multiple_of(base, 32)
    pltpu.sync_copy(an_hbm.at[pl.ds(base, BPW)], idx_v)

    def chunk(j, carry):
        off = pl.multiple_of(j * CH, CH)
        pltpu.async_copy(
            table_hbm.at[idx_v.at[pl.ds(off, CH)]],
            rows_v.at[pl.ds(off, CH)],
            sem,
        )
        return carry

    lax.fori_loop(0, NCH, chunk, 0)
    pltpu.make_async_copy(out_hbm.at[pl.ds(base, BPW)], rows_v, sem).wait()
    pltpu.sync_copy(rows_v, out_hbm.at[pl.ds(base, BPW)])


def kernel(atomic_number, group_mapping, row_mapping):
    table = _build_table(group_mapping, row_mapping)
    return _gather(atomic_number, table)


# speedup vs baseline: 11.8295x; 1.4673x over previous
import functools

import jax
import jax.numpy as jnp
from jax import lax
from jax.experimental import pallas as pl
from jax.experimental.pallas import tpu as pltpu
from jax.experimental.pallas import tpu_sc as plsc

N = 100000
D = 24
V = 84
VP = 96

_info = plsc.get_sparse_core_info()
_NC, _NS = _info.num_cores, _info.num_subcores
NW = _NC * _NS
BPW = 3200
NCH = BPW // 16
LAST_BASE = N - BPW


def _cols_body(gm_ref, rm_ref, cols_ref):
    g = gm_ref[:]
    r = rm_ref[:]
    cols_ref[:] = jnp.concatenate([g, r + 18], axis=0)


def _build_cols(gm, rm):
    gmp = jnp.pad(gm[:V], (0, VP - V)).reshape(VP, 1)
    rmp = jnp.pad(rm[:V], (0, VP - V)).reshape(VP, 1)
    return pl.pallas_call(
        _cols_body,
        out_shape=jax.ShapeDtypeStruct((2 * VP, 1), jnp.int32),
    )(gmp, rmp).reshape(2 * VP)


_mesh = plsc.VectorSubcoreMesh(core_axis_name="c", subcore_axis_name="s")


@functools.partial(
    pl.kernel,
    mesh=_mesh,
    out_type=jax.ShapeDtypeStruct((N * D,), jnp.float32),
    scratch_types=[
        pltpu.VMEM((BPW,), jnp.int32),
        pltpu.VMEM((2 * VP,), jnp.int32),
        pltpu.VMEM((BPW * D,), jnp.float32),
    ],
    compiler_params=pltpu.CompilerParams(
        use_tc_tiling_on_sc=False, needs_layout_passes=False
    ),
)
def _scatter_onehot(an_hbm, cols_hbm, out_hbm, idx_v, cols_v, rows_v):
    wid = lax.axis_index("s") * _NC + lax.axis_index("c")
    base = jnp.minimum(wid * BPW, LAST_BASE)
    base = pl.multiple_of(base, 32)
    pltpu.sync_copy(an_hbm.at[pl.ds(base, BPW)], idx_v)
    pltpu.sync_copy(cols_hbm, cols_v)

    lane24 = lax.iota(jnp.int32, 16) * D
    zf = jnp.zeros((16,), jnp.float32)
    onef = jnp.ones((16,), jnp.float32)

    def chunk(c, carry):
        o = pl.multiple_of(c * (16 * D), 16 * D)
        for k in range(D):
            rows_v[pl.ds(o + k * 16, 16)] = zf
        v = idx_v[pl.ds(pl.multiple_of(c * 16, 16), 16)]
        c1 = plsc.load_gather(cols_v, [v])
        c2 = plsc.load_gather(cols_v, [v + VP])
        addr = o + lane24
        plsc.store_scatter(rows_v, [addr + c1], onef)
        plsc.store_scatter(rows_v, [addr + c2], onef)
        return carry

    lax.fori_loop(0, NCH, chunk, 0)
    out_off = pl.multiple_of(base * D, 32 * D)
    pltpu.sync_copy(rows_v, out_hbm.at[pl.ds(out_off, BPW * D)])


def kernel(atomic_number, group_mapping, row_mapping):
    cols = _build_cols(group_mapping, row_mapping)
    flat = _scatter_onehot(atomic_number, cols)
    return flat.reshape(N, D)


# speedup vs baseline: 14.7285x; 1.2451x over previous
import functools

import jax
import jax.numpy as jnp
from jax import lax
from jax.experimental import pallas as pl
from jax.experimental.pallas import tpu as pltpu
from jax.experimental.pallas import tpu_sc as plsc

N = 100000
D = 24
V = 84
VP = 96

_info = plsc.get_sparse_core_info()
_NC, _NS = _info.num_cores, _info.num_subcores
NW = _NC * _NS
BPW = 3200
SLAB = 800
NSL = BPW // SLAB
NCH = SLAB // 16
LAST_BASE = N - BPW


def _cols_body(gm_ref, rm_ref, cols_ref):
    g = gm_ref[:]
    r = rm_ref[:]
    cols_ref[:] = jnp.concatenate([g, r + 18], axis=0)


def _build_cols(gm, rm):
    gmp = jnp.pad(gm[:V], (0, VP - V)).reshape(VP, 1)
    rmp = jnp.pad(rm[:V], (0, VP - V)).reshape(VP, 1)
    return pl.pallas_call(
        _cols_body,
        out_shape=jax.ShapeDtypeStruct((2 * VP, 1), jnp.int32),
    )(gmp, rmp).reshape(2 * VP)


_mesh = plsc.VectorSubcoreMesh(core_axis_name="c", subcore_axis_name="s")


@functools.partial(
    pl.kernel,
    mesh=_mesh,
    out_type=jax.ShapeDtypeStruct((N, D), jnp.float32),
    scratch_types=[
        pltpu.VMEM((BPW,), jnp.int32),
        pltpu.VMEM((2 * VP,), jnp.int32),
        pltpu.VMEM((SLAB, D), jnp.float32),
    ],
    compiler_params=pltpu.CompilerParams(
        use_tc_tiling_on_sc=True, needs_layout_passes=False
    ),
)
def _scatter_onehot(an_hbm, cols_hbm, out_hbm, idx_v, cols_v, rows_v):
    wid = lax.axis_index("s") * _NC + lax.axis_index("c")
    base = jnp.minimum(wid * BPW, LAST_BASE)
    base = pl.multiple_of(base, 32)
    pltpu.sync_copy(an_hbm.at[pl.ds(base, BPW)], idx_v)
    pltpu.sync_copy(cols_hbm, cols_v)

    lane = lax.iota(jnp.int32, 16)
    zf = jnp.zeros((16,), jnp.float32)
    onef = jnp.ones((16,), jnp.float32)

    for sl in range(NSL):
        def chunk(c, carry):
            r0 = c * 16
            for j in range(16):
                r = jnp.full((16,), r0 + j, jnp.int32)
                plsc.store_scatter(rows_v, [r, lane], zf)
                plsc.store_scatter(rows_v, [r, lane + (D - 16)], zf)
            v = idx_v[pl.ds(pl.multiple_of(sl * SLAB + r0, 16), 16)]
            c1 = plsc.load_gather(cols_v, [v])
            c2 = plsc.load_gather(cols_v, [v + VP])
            rvec = r0 + lane
            plsc.store_scatter(rows_v, [rvec, c1], onef)
            plsc.store_scatter(rows_v, [rvec, c2], onef)
            return carry

        lax.fori_loop(0, NCH, chunk, 0)
        pltpu.sync_copy(rows_v, out_hbm.at[pl.ds(base + sl * SLAB, SLAB)])


def kernel(atomic_number, group_mapping, row_mapping):
    cols = _build_cols(group_mapping, row_mapping)
    return _scatter_onehot(atomic_number, cols)


# speedup vs baseline: 29.2749x; 1.9876x over previous
import functools

import jax
import jax.numpy as jnp
from jax import lax
from jax.experimental import pallas as pl
from jax.experimental.pallas import tpu as pltpu
from jax.experimental.pallas import tpu_sc as plsc

N = 100000
D = 24
V = 84
VP = 96

_info = plsc.get_sparse_core_info()
_NC, _NS = _info.num_cores, _info.num_subcores
NW = _NC * _NS
NP = 100096
SLAB = 1024
NFULL = NP // SLAB
TAIL = NP - NFULL * SLAB
TVAL = N - NFULL * SLAB
KMAIN = NFULL // NW


def _cols_body(gm_ref, rm_ref, cols_ref):
    g = gm_ref[:]
    r = rm_ref[:]
    cols_ref[:] = jnp.concatenate([g, r + 18], axis=0)


def _build_cols(gm, rm):
    gmp = jnp.pad(gm[:V], (0, VP - V)).reshape(VP, 1)
    rmp = jnp.pad(rm[:V], (0, VP - V)).reshape(VP, 1)
    return pl.pallas_call(
        _cols_body,
        out_shape=jax.ShapeDtypeStruct((2 * VP, 1), jnp.int32),
    )(gmp, rmp).reshape(2 * VP)


_mesh = plsc.VectorSubcoreMesh(core_axis_name="c", subcore_axis_name="s")


@functools.partial(
    pl.kernel,
    mesh=_mesh,
    out_type=jax.ShapeDtypeStruct((D, N), jnp.float32),
    scratch_types=[
        pltpu.VMEM((SLAB,), jnp.int32),
        pltpu.VMEM((2 * VP,), jnp.int32),
        pltpu.VMEM((D, SLAB), jnp.float32),
    ],
    compiler_params=pltpu.CompilerParams(
        use_tc_tiling_on_sc=True,
        needs_layout_passes=False,
        disable_bounds_checks=True,
    ),
)
def _scatter_onehot(an_hbm, cols_hbm, out_hbm, idx_v, cols_v, rows_v):
    wid = lax.axis_index("s") * _NC + lax.axis_index("c")
    pltpu.sync_copy(cols_hbm, cols_v)

    lane = lax.iota(jnp.int32, 16)
    zf = jnp.zeros((16,), jnp.float32)
    onef = jnp.ones((16,), jnp.float32)

    def do_slab(base, width, valid):
        base = pl.multiple_of(base, 128)
        pltpu.sync_copy(
            an_hbm.at[pl.ds(base, valid)], idx_v.at[pl.ds(0, valid)]
        )

        def zero_chunk(c, carry):
            ivec = c * 16 + lane
            for j in range(D):
                plsc.store_scatter(rows_v, [jnp.full((16,), j, jnp.int32), ivec], zf)
            return carry

        def ones_chunk(c, carry):
            o = c * 16
            v = idx_v[pl.ds(pl.multiple_of(o, 16), 16)]
            c1 = plsc.load_gather(cols_v, [v])
            c2 = plsc.load_gather(cols_v, [v + VP])
            ivec = o + lane
            plsc.store_scatter(rows_v, [c1, ivec], onef)
            plsc.store_scatter(rows_v, [c2, ivec], onef)
            return carry

        lax.fori_loop(0, width // 16, zero_chunk, 0)
        lax.fori_loop(0, valid // 16, ones_chunk, 0)
        pltpu.sync_copy(
            rows_v.at[:, pl.ds(0, width)], out_hbm.at[:, pl.ds(base, width)]
        )

    for k in range(KMAIN):
        do_slab((wid + NW * k) * SLAB, SLAB, SLAB)

    @pl.when(wid == 0)
    def _():
        do_slab(KMAIN * NW * SLAB, SLAB, SLAB)

    @pl.when(wid == 1)
    def _():
        do_slab(NFULL * SLAB, TAIL, TVAL)


def kernel(atomic_number, group_mapping, row_mapping):
    cols = _build_cols(group_mapping, row_mapping)
    out_t = _scatter_onehot(atomic_number, cols)
    return out_t.T
